# Initial kernel scaffold; baseline (speedup 1.0000x reference)
import jax
import jax.numpy as jnp
from jax.experimental import pallas as pl


def kernel(positions, atomic_numbers, edge_index, shifts, node_embedding_W, radial_transform_W):
    raise NotImplementedError("write your pallas kernel here")


# speedup vs baseline: 11.6969x; 11.6969x over previous
import functools
import math

import jax
import jax.numpy as jnp
from jax import lax
from jax.experimental import pallas as pl
from jax.experimental.pallas import tpu as pltpu
from jax.experimental.pallas import tpu_sc as plsc

N_NODES = 10000
N_EDGES = 160000
N_RBF = 8
CUTOFF = 5.5
ZS_VALS = (1, 6, 7, 8)

_LXLYLZ = [(0, 0, 0),
           (1, 0, 0), (0, 1, 0), (0, 0, 1),
           (2, 0, 0), (1, 1, 0), (1, 0, 1), (0, 2, 0), (0, 1, 1), (0, 0, 2)]
_L_OF = [0, 1, 1, 1, 2, 2, 2, 2, 2, 2]
_PREF2 = [1.0, 2.0, 2.0, 1.0, 2.0, 1.0]

_CHUNK = 128
_N_CHUNKS = N_EDGES // _CHUNK
_STRIPE = N_NODES // 16
_EB = 640
_NB = 400


def _table_body(pos_ref, z_ref, w_ref, out_ref):
    z = z_ref[...]
    col = lax.broadcasted_iota(jnp.int32, (1, 4), 1)
    zs = ((col == 0) * ZS_VALS[0] + (col == 1) * ZS_VALS[1]
          + (col == 2) * ZS_VALS[2] + (col == 3) * ZS_VALS[3])
    one_hot = (z == zs).astype(jnp.float32)
    emb = jax.lax.dot_general(one_hot, w_ref[...], (((1,), (0,)), ((), ())),
                              precision=lax.Precision.HIGHEST)
    out_ref[:, 0:3] = pos_ref[...]
    out_ref[:, 3:5] = emb
    out_ref[:, 5:16] = jnp.zeros((z.shape[0], 11), jnp.float32)


def _build_table(positions, z2d, node_embedding_W):
    return pl.pallas_call(
        _table_body,
        out_shape=jax.ShapeDtypeStruct((N_NODES, 16), jnp.float32),
    )(positions, z2d, node_embedding_W)


def _gather_body(table_hbm, idx_hbm, out_hbm, idx_v, rows_v, sem):
    cid = lax.axis_index("c")
    sid = lax.axis_index("s")
    wid = sid * 2 + cid
    n_rows = 2 * _N_CHUNKS

    def step(k, carry):
        cc = wid + 32 * k

        @pl.when(cc < n_rows)
        def _():
            pltpu.sync_copy(idx_hbm.at[cc], idx_v)
            pltpu.async_copy(table_hbm.at[idx_v], rows_v, sem).wait()
            pltpu.sync_copy(rows_v, out_hbm.at[cc])
        return carry

    lax.fori_loop(0, (n_rows + 31) // 32, step, 0)


def _sc_gather(table, idx2d):
    f = pl.kernel(
        _gather_body,
        out_type=jax.ShapeDtypeStruct((2 * _N_CHUNKS, _CHUNK, 16),
                                      jnp.float32),
        mesh=plsc.VectorSubcoreMesh(core_axis_name="c", subcore_axis_name="s"),
        scratch_types=[
            pltpu.VMEM((_CHUNK,), jnp.int32),
            pltpu.VMEM((_CHUNK, 16), jnp.float32),
            pltpu.SemaphoreType.DMA,
        ],
        compiler_params=pltpu.CompilerParams(use_tc_tiling_on_sc=False),
    )
    return f(table, idx2d)


def _edge_body(g_ref, shifts_ref, w_ref, out_ref):
    gs = g_ref[0]
    gr = g_ref[1]
    vec = gr[:, 0:3] - gs[:, 0:3] + shifts_ref[...]
    d2 = jnp.sum(vec * vec, axis=1, keepdims=True)
    lengths = jnp.sqrt(d2 + 1e-12)
    inv_len = 1.0 / lengths
    unit = vec * inv_len

    k_pi = ((lax.broadcasted_iota(jnp.int32, (1, N_RBF), 1) + 1)
            .astype(jnp.float32) * math.pi)
    rbf = (math.sqrt(2.0 / CUTOFF)
           * jnp.sin(k_pi * (lengths / CUTOFF)) * inv_len)
    r = lengths / CUTOFF
    r2 = r * r
    r6 = r2 * r2 * r2
    r7 = r6 * r
    r8 = r7 * r
    fc = 1.0 - 28.0 * r6 + 48.0 * r7 - 21.0 * r8
    fc = jnp.where(lengths < CUTOFF, fc, 0.0)
    radial = rbf * fc

    wcat = jnp.concatenate([w_ref[0], w_ref[1], w_ref[2]], axis=1)
    rl = jax.lax.dot_general(radial, wcat, (((1,), (0,)), ((), ())),
                             precision=lax.Precision.HIGHEST)

    x = unit[:, 0:1]
    y = unit[:, 1:2]
    z = unit[:, 2:3]
    one = jnp.ones_like(x)
    ang = jnp.concatenate(
        [one, x, y, z, x * x, x * y, x * z, y * y, y * z, z * z], axis=1)

    es0 = gs[:, 3:4]
    es1 = gs[:, 4:5]
    er = gr[:, 3:5]
    enc = jnp.concatenate([es0 * er, es1 * er], axis=1)

    m = lax.broadcasted_iota(jnp.int32, (1, 320), 1)
    a_m = m // 32
    s_m = (m % 32) // 4
    c_m = m % 4
    l_m = (a_m >= 1).astype(jnp.int32) + (a_m >= 4).astype(jnp.int32)

    r1 = lax.broadcasted_iota(jnp.int32, (24, 320), 0)
    m1 = (r1 == (l_m * 8 + s_m)).astype(jnp.float32)
    r2i = lax.broadcasted_iota(jnp.int32, (10, 320), 0)
    m2 = (r2i == a_m).astype(jnp.float32)
    r3i = lax.broadcasted_iota(jnp.int32, (4, 320), 0)
    m3 = (r3i == c_m).astype(jnp.float32)

    dot = functools.partial(jax.lax.dot_general,
                            dimension_numbers=(((1,), (0,)), ((), ())),
                            precision=lax.Precision.HIGHEST)
    g = dot(rl, m1) * dot(ang, m2) * dot(enc, m3)
    out_ref[0] = g[:, 0:160]
    out_ref[1] = g[:, 160:320]


def _edge_expand(g, shifts, radial_transform_W):
    grid = N_EDGES // _EB
    return pl.pallas_call(
        _edge_body,
        grid=(grid,),
        in_specs=[
            pl.BlockSpec((2, _EB, 16), lambda i: (0, i, 0)),
            pl.BlockSpec((_EB, 3), lambda i: (i, 0)),
            pl.BlockSpec((3, 8, 8), lambda i: (0, 0, 0)),
        ],
        out_specs=pl.BlockSpec((2, _EB, 160), lambda i: (0, i, 0)),
        out_shape=jax.ShapeDtypeStruct((2, N_EDGES, 160), jnp.float32),
    )(g, shifts, radial_transform_W)


def _scatter_body(exp_hbm, recv_hbm, zer_hbm, out_hbm, idx_v, rows_v, acc):
    cid = lax.axis_index("c")
    tid = lax.axis_index("s")

    pltpu.sync_copy(zer_hbm, acc.at[pl.ds(tid * _STRIPE, _STRIPE), :])
    plsc.subcore_barrier()

    def step(k, carry):
        c = tid + 16 * k

        @pl.when(c < _N_CHUNKS)
        def _():
            pltpu.sync_copy(recv_hbm.at[c], idx_v)
            pltpu.sync_copy(exp_hbm.at[cid, pl.ds(c * _CHUNK, _CHUNK), :],
                            rows_v)
            pltpu.sync_copy(rows_v, acc.at[idx_v], add=True)
        return carry

    lax.fori_loop(0, (_N_CHUNKS + 15) // 16, step, 0)
    plsc.subcore_barrier()
    pltpu.sync_copy(acc.at[pl.ds(tid * _STRIPE, _STRIPE), :],
                    out_hbm.at[cid, pl.ds(tid * _STRIPE, _STRIPE), :])


def _sc_scatter(expanded, recv2d, zer):
    f = pl.kernel(
        _scatter_body,
        out_type=jax.ShapeDtypeStruct((2, N_NODES, 160), jnp.float32),
        mesh=plsc.VectorSubcoreMesh(core_axis_name="c", subcore_axis_name="s"),
        scratch_types=[
            pltpu.VMEM((_CHUNK,), jnp.int32),
            pltpu.VMEM((_CHUNK, 160), jnp.float32),
            pltpu.VMEM_SHARED((N_NODES, 160), jnp.float32),
        ],
        compiler_params=pltpu.CompilerParams(use_tc_tiling_on_sc=False),
    )
    return f(expanded, recv2d, zer)


def _sym_body(nf_ref, out_ref):
    h0 = nf_ref[0]
    h1 = nf_ref[1]
    nu1 = h0[:, 0:32]
    nu21 = (h0[:, 32:64] * h0[:, 32:64]
            + h0[:, 64:96] * h0[:, 64:96]
            + h0[:, 96:128] * h0[:, 96:128])
    a4 = h0[:, 128:160]
    nu22 = _PREF2[0] * a4 * a4
    for j, pref in enumerate(_PREF2[1:]):
        blk = h1[:, 32 * j:32 * j + 32]
        nu22 = nu22 + pref * blk * blk
    pieces = []
    for s in range(8):
        pieces.append(nu1[:, 4 * s:4 * s + 4])
        pieces.append(nu21[:, 4 * s:4 * s + 4])
        pieces.append(nu22[:, 4 * s:4 * s + 4])
    out_ref[...] = jnp.concatenate(pieces, axis=1)


def _symmetrize(nfa):
    grid = N_NODES // _NB
    return pl.pallas_call(
        _sym_body,
        grid=(grid,),
        in_specs=[pl.BlockSpec((2, _NB, 160), lambda i: (0, i, 0))],
        out_specs=pl.BlockSpec((_NB, 96), lambda i: (i, 0)),
        out_shape=jax.ShapeDtypeStruct((N_NODES, 96), jnp.float32),
    )(nfa)


def kernel(positions, atomic_numbers, edge_index, shifts,
           node_embedding_W, radial_transform_W):
    n = positions.shape[0]
    z2d = atomic_numbers.reshape(n, 1).astype(jnp.int32)
    table = _build_table(positions, z2d, node_embedding_W)

    idx2d = edge_index.astype(jnp.int32).reshape(2 * _N_CHUNKS, _CHUNK)
    gathered = _sc_gather(table, idx2d)
    g = gathered.reshape(2, N_EDGES, 16)

    expanded = _edge_expand(g, shifts, radial_transform_W)

    recv2d = edge_index[1].astype(jnp.int32).reshape(_N_CHUNKS, _CHUNK)
    zer = jnp.zeros((_STRIPE, 160), jnp.float32)
    nfa = _sc_scatter(expanded, recv2d, zer)

    out96 = _symmetrize(nfa)
    return out96.reshape(n, 8, 3, 4)


# speedup vs baseline: 21.3980x; 1.8294x over previous
import functools
import math

import jax
import jax.numpy as jnp
import numpy as np
from jax import lax
from jax.experimental import pallas as pl
from jax.experimental.pallas import tpu as pltpu
from jax.experimental.pallas import tpu_sc as plsc

N_NODES = 10000
N_EDGES = 160000
N_RBF = 8
CUTOFF = 5.5
ZS_VALS = (1, 6, 7, 8)

_LXLYLZ = [(0, 0, 0),
           (1, 0, 0), (0, 1, 0), (0, 0, 1),
           (2, 0, 0), (1, 1, 0), (1, 0, 1), (0, 2, 0), (0, 1, 1), (0, 0, 2)]
_L_OF = [0, 1, 1, 1, 2, 2, 2, 2, 2, 2]
_PREF2 = [1.0, 2.0, 2.0, 1.0, 2.0, 1.0]

_CHUNK = 128
_N_CHUNKS = N_EDGES // _CHUNK
_STRIPE = N_NODES // 16
_EB = 640
_NB = 400


def _table_body(pos_ref, z_ref, w_ref, out_ref):
    z = z_ref[...]
    col = lax.broadcasted_iota(jnp.int32, (1, 4), 1)
    zs = ((col == 0) * ZS_VALS[0] + (col == 1) * ZS_VALS[1]
          + (col == 2) * ZS_VALS[2] + (col == 3) * ZS_VALS[3])
    one_hot = (z == zs).astype(jnp.float32)
    emb = jax.lax.dot_general(one_hot, w_ref[...], (((1,), (0,)), ((), ())),
                              precision=lax.Precision.HIGHEST)
    out_ref[:, 0:3] = pos_ref[...]
    out_ref[:, 3:5] = emb
    out_ref[:, 5:16] = jnp.zeros((z.shape[0], 11), jnp.float32)


def _build_table(positions, z2d, node_embedding_W):
    return pl.pallas_call(
        _table_body,
        out_shape=jax.ShapeDtypeStruct((N_NODES, 16), jnp.float32),
    )(positions, z2d, node_embedding_W)


def _gather_body(table_hbm, idx_hbm, out_hbm, idx_v, rows_v, sem):
    cid = lax.axis_index("c")
    sid = lax.axis_index("s")
    wid = sid * 2 + cid
    n_rows = 2 * _N_CHUNKS

    def step(k, carry):
        cc = wid + 32 * k

        @pl.when(cc < n_rows)
        def _():
            pltpu.sync_copy(idx_hbm.at[cc], idx_v)
            pltpu.async_copy(table_hbm.at[idx_v], rows_v, sem).wait()
            pltpu.sync_copy(rows_v, out_hbm.at[cc])
        return carry

    lax.fori_loop(0, (n_rows + 31) // 32, step, 0)


def _sc_gather(table, idx2d):
    f = pl.kernel(
        _gather_body,
        out_type=jax.ShapeDtypeStruct((2 * _N_CHUNKS, _CHUNK, 16),
                                      jnp.float32),
        mesh=plsc.VectorSubcoreMesh(core_axis_name="c", subcore_axis_name="s"),
        scratch_types=[
            pltpu.VMEM((_CHUNK,), jnp.int32),
            pltpu.VMEM((_CHUNK, 16), jnp.float32),
            pltpu.SemaphoreType.DMA,
        ],
        compiler_params=pltpu.CompilerParams(use_tc_tiling_on_sc=False),
    )
    return f(table, idx2d)


def _np_masks():
    col = np.arange(320)
    a_m = col // 32
    s_m = (col % 32) // 4
    c_m = col % 4
    l_m = (a_m >= 1).astype(np.int32) + (a_m >= 4).astype(np.int32)
    m1 = (np.arange(24)[:, None] == (l_m * 8 + s_m)[None, :]
          ).astype(np.float32)
    m23 = (np.arange(40)[:, None] == (a_m * 4 + c_m)[None, :]
           ).astype(np.float32)
    return m1, m23


def _edge_body(g_ref, shifts_ref, w_ref, m1_ref, m23_ref, out_ref):
    gst = jnp.transpose(g_ref[0])
    grt = jnp.transpose(g_ref[1])
    vec = grt[0:3] - gst[0:3] + shifts_ref[...]
    d2 = jnp.sum(vec * vec, axis=0, keepdims=True)
    lengths = jnp.sqrt(d2 + 1e-12)
    inv_len = 1.0 / lengths
    unit = vec * inv_len

    r = lengths * (1.0 / CUTOFF)
    r2 = r * r
    r6 = r2 * r2 * r2
    r7 = r6 * r
    r8 = r7 * r
    fc = 1.0 - 28.0 * r6 + 48.0 * r7 - 21.0 * r8
    fc = jnp.where(lengths < CUTOFF, fc, 0.0)
    pref = fc * (math.sqrt(2.0 / CUTOFF)) * inv_len

    k_col = (lax.broadcasted_iota(jnp.int32, (N_RBF, 1), 0) + 1
             ).astype(jnp.float32)
    theta_t = jax.lax.dot_general(
        k_col, lengths * (math.pi / CUTOFF), (((1,), (0,)), ((), ())),
        precision=lax.Precision.HIGHEST)
    radial_t = jnp.sin(theta_t)

    x = unit[0:1]
    y = unit[1:2]
    z = unit[2:3]
    one = jnp.ones_like(x)
    es0 = gst[3:4]
    es1 = gst[4:5]
    er0 = grt[3:4]
    er1 = grt[4:5]
    enct = jnp.concatenate([es0 * er0, es0 * er1, es1 * er0, es1 * er1],
                           axis=0) * pref
    monos = [one, x, y, z, x * x, x * y, x * z, y * y, y * z, z * z]
    ae_t = jnp.concatenate([mono * enct for mono in monos], axis=0)

    wcat = jnp.concatenate([w_ref[0], w_ref[1], w_ref[2]], axis=1)
    w320 = jax.lax.dot_general(wcat, m1_ref[...], (((1,), (0,)), ((), ())),
                               precision=lax.Precision.HIGHEST)

    dot0 = functools.partial(jax.lax.dot_general,
                             dimension_numbers=(((0,), (0,)), ((), ())),
                             precision=lax.Precision.DEFAULT)

    def bf16_split(a):
        a_hi = a.astype(jnp.bfloat16).astype(jnp.float32)
        return a_hi, a - a_hi

    r_hi, r_lo = bf16_split(radial_t)
    w_hi, w_lo = bf16_split(w320)
    g1 = dot0(r_hi, w_hi) + dot0(r_hi, w_lo) + dot0(r_lo, w_hi)
    ae_hi, ae_lo = bf16_split(ae_t)
    gae = dot0(ae_hi, m23_ref[...]) + dot0(ae_lo, m23_ref[...])
    out_ref[...] = g1 * gae


def _edge_expand(g, shifts_t, radial_transform_W):
    grid = N_EDGES // _EB
    m1, m23 = _np_masks()
    return pl.pallas_call(
        _edge_body,
        grid=(grid,),
        in_specs=[
            pl.BlockSpec((2, _EB, 16), lambda i: (0, i, 0)),
            pl.BlockSpec((3, _EB), lambda i: (0, i)),
            pl.BlockSpec((3, 8, 8), lambda i: (0, 0, 0)),
            pl.BlockSpec((24, 320), lambda i: (0, 0)),
            pl.BlockSpec((40, 320), lambda i: (0, 0)),
        ],
        out_specs=pl.BlockSpec((_EB, 320), lambda i: (i, 0)),
        out_shape=jax.ShapeDtypeStruct((N_EDGES, 320), jnp.float32),
    )(g, shifts_t, radial_transform_W, m1, m23)


def _scatter_body(exp_hbm, recv_hbm, zer_hbm, out_hbm, idx_v, rows_v, acc):
    cid = lax.axis_index("c")
    tid = lax.axis_index("s")

    pltpu.sync_copy(zer_hbm, acc.at[pl.ds(tid * _STRIPE, _STRIPE), :])
    plsc.subcore_barrier()

    def step(k, carry):
        c = tid + 16 * k

        @pl.when(c < _N_CHUNKS)
        def _():
            pltpu.sync_copy(recv_hbm.at[c], idx_v)
            pltpu.sync_copy(exp_hbm.at[pl.ds(c * _CHUNK, _CHUNK),
                                       pl.ds(cid * 160, 160)],
                            rows_v)
            pltpu.sync_copy(rows_v, acc.at[idx_v], add=True)
        return carry

    lax.fori_loop(0, (_N_CHUNKS + 15) // 16, step, 0)
    plsc.subcore_barrier()
    pltpu.sync_copy(acc.at[pl.ds(tid * _STRIPE, _STRIPE), :],
                    out_hbm.at[cid, pl.ds(tid * _STRIPE, _STRIPE), :])


def _sc_scatter(expanded, recv2d, zer):
    f = pl.kernel(
        _scatter_body,
        out_type=jax.ShapeDtypeStruct((2, N_NODES, 160), jnp.float32),
        mesh=plsc.VectorSubcoreMesh(core_axis_name="c", subcore_axis_name="s"),
        scratch_types=[
            pltpu.VMEM((_CHUNK,), jnp.int32),
            pltpu.VMEM((_CHUNK, 160), jnp.float32),
            pltpu.VMEM_SHARED((N_NODES, 160), jnp.float32),
        ],
        compiler_params=pltpu.CompilerParams(use_tc_tiling_on_sc=False),
    )
    return f(expanded, recv2d, zer)


def _sym_body(nf_ref, out_ref):
    h0 = nf_ref[0]
    h1 = nf_ref[1]
    nu1 = h0[:, 0:32]
    nu21 = (h0[:, 32:64] * h0[:, 32:64]
            + h0[:, 64:96] * h0[:, 64:96]
            + h0[:, 96:128] * h0[:, 96:128])
    a4 = h0[:, 128:160]
    nu22 = _PREF2[0] * a4 * a4
    for j, pref in enumerate(_PREF2[1:]):
        blk = h1[:, 32 * j:32 * j + 32]
        nu22 = nu22 + pref * blk * blk
    pieces = []
    for s in range(8):
        pieces.append(nu1[:, 4 * s:4 * s + 4])
        pieces.append(nu21[:, 4 * s:4 * s + 4])
        pieces.append(nu22[:, 4 * s:4 * s + 4])
    out_ref[...] = jnp.concatenate(pieces, axis=1)


def _symmetrize(nfa):
    grid = N_NODES // _NB
    return pl.pallas_call(
        _sym_body,
        grid=(grid,),
        in_specs=[pl.BlockSpec((2, _NB, 160), lambda i: (0, i, 0))],
        out_specs=pl.BlockSpec((_NB, 96), lambda i: (i, 0)),
        out_shape=jax.ShapeDtypeStruct((N_NODES, 96), jnp.float32),
    )(nfa)


def kernel(positions, atomic_numbers, edge_index, shifts,
           node_embedding_W, radial_transform_W):
    n = positions.shape[0]
    z2d = atomic_numbers.reshape(n, 1).astype(jnp.int32)
    table = _build_table(positions, z2d, node_embedding_W)

    idx2d = edge_index.astype(jnp.int32).reshape(2 * _N_CHUNKS, _CHUNK)
    gathered = _sc_gather(table, idx2d)
    g = gathered.reshape(2, N_EDGES, 16)

    expanded = _edge_expand(g, shifts.T, radial_transform_W)

    recv2d = edge_index[1].astype(jnp.int32).reshape(_N_CHUNKS, _CHUNK)
    zer = jnp.zeros((_STRIPE, 160), jnp.float32)
    nfa = _sc_scatter(expanded, recv2d, zer)

    out96 = _symmetrize(nfa)
    return out96.reshape(n, 8, 3, 4)


# speedup vs baseline: 27.5772x; 1.2888x over previous
import functools
import math

import jax
import jax.numpy as jnp
import numpy as np
from jax import lax
from jax.experimental import pallas as pl
from jax.experimental.pallas import tpu as pltpu
from jax.experimental.pallas import tpu_sc as plsc

N_NODES = 10000
N_EDGES = 160000
N_RBF = 8
CUTOFF = 5.5
ZS_VALS = (1, 6, 7, 8)

_LXLYLZ = [(0, 0, 0),
           (1, 0, 0), (0, 1, 0), (0, 0, 1),
           (2, 0, 0), (1, 1, 0), (1, 0, 1), (0, 2, 0), (0, 1, 1), (0, 0, 2)]
_L_OF = [0, 1, 1, 1, 2, 2, 2, 2, 2, 2]
_PREF2 = [1.0, 2.0, 2.0, 1.0, 2.0, 1.0]

_CHUNK = 128
_N_CHUNKS = N_EDGES // _CHUNK
_STRIPE = N_NODES // 16
_EB = 640
_NB = 400


def _table_body(pos_ref, z_ref, w_ref, out_ref):
    z = z_ref[...]
    col = lax.broadcasted_iota(jnp.int32, (1, 4), 1)
    zs = ((col == 0) * ZS_VALS[0] + (col == 1) * ZS_VALS[1]
          + (col == 2) * ZS_VALS[2] + (col == 3) * ZS_VALS[3])
    one_hot = (z == zs).astype(jnp.float32)
    emb = jax.lax.dot_general(one_hot, w_ref[...], (((1,), (0,)), ((), ())),
                              precision=lax.Precision.HIGHEST)
    out_ref[:, 0:3] = pos_ref[...]
    out_ref[:, 3:5] = emb
    out_ref[:, 5:16] = jnp.zeros((z.shape[0], 11), jnp.float32)


def _build_table(positions, z2d, node_embedding_W):
    return pl.pallas_call(
        _table_body,
        out_shape=jax.ShapeDtypeStruct((N_NODES, 16), jnp.float32),
    )(positions, z2d, node_embedding_W)


def _gather_body(table_hbm, idx_hbm, out_hbm, idx_v, rows_v, trows_v, sem):
    cid = lax.axis_index("c")
    sid = lax.axis_index("s")
    wid = sid * 2 + cid
    n_rows = 2 * _N_CHUNKS
    lane = lax.iota(jnp.int32, 16)

    def step(k, carry):
        cc = wid + 32 * k

        @pl.when(cc < n_rows)
        def _():
            pltpu.sync_copy(idx_hbm.at[cc], idx_v)
            pltpu.async_copy(table_hbm.at[idx_v], rows_v, sem).wait()
            for t in range(16):
                t_idx = jnp.full((16,), t, jnp.int32)
                for gi in range(8):
                    vals = plsc.load_gather(rows_v, [lane + 16 * gi, t_idx])
                    trows_v[t, pl.ds(16 * gi, 16)] = vals
            pltpu.sync_copy(trows_v, out_hbm.at[cc])
        return carry

    lax.fori_loop(0, (n_rows + 31) // 32, step, 0)


def _sc_gather(table, idx2d):
    f = pl.kernel(
        _gather_body,
        out_type=jax.ShapeDtypeStruct((2 * _N_CHUNKS, 16, _CHUNK),
                                      jnp.float32),
        mesh=plsc.VectorSubcoreMesh(core_axis_name="c", subcore_axis_name="s"),
        scratch_types=[
            pltpu.VMEM((_CHUNK,), jnp.int32),
            pltpu.VMEM((_CHUNK, 16), jnp.float32),
            pltpu.VMEM((16, _CHUNK), jnp.float32),
            pltpu.SemaphoreType.DMA,
        ],
        compiler_params=pltpu.CompilerParams(use_tc_tiling_on_sc=False,
                                             needs_layout_passes=False),
    )
    return f(table, idx2d)


def _np_masks():
    col = np.arange(320)
    a_m = col // 32
    s_m = (col % 32) // 4
    c_m = col % 4
    l_m = (a_m >= 1).astype(np.int32) + (a_m >= 4).astype(np.int32)
    m1 = (np.arange(24)[:, None] == (l_m * 8 + s_m)[None, :]
          ).astype(np.float32)
    m23 = (np.arange(40)[:, None] == (a_m * 4 + c_m)[None, :]
           ).astype(np.float32)
    return m1, m23


def _edge_body(gs_ref, gr_ref, shifts_ref, w_ref, m1_ref, m23_ref,
               out_a, out_b, out_c):
    nch = _EB // _CHUNK
    gst = jnp.concatenate([gs_ref[i] for i in range(nch)], axis=1)
    grt = jnp.concatenate([gr_ref[i] for i in range(nch)], axis=1)
    vec = grt[0:3] - gst[0:3] + shifts_ref[...]
    d2 = jnp.sum(vec * vec, axis=0, keepdims=True)
    lengths = jnp.sqrt(d2 + 1e-12)
    inv_len = 1.0 / lengths
    unit = vec * inv_len

    r = lengths * (1.0 / CUTOFF)
    r2 = r * r
    r6 = r2 * r2 * r2
    r7 = r6 * r
    r8 = r7 * r
    fc = 1.0 - 28.0 * r6 + 48.0 * r7 - 21.0 * r8
    fc = jnp.where(lengths < CUTOFF, fc, 0.0)
    pref = fc * (math.sqrt(2.0 / CUTOFF)) * inv_len

    k_col = (lax.broadcasted_iota(jnp.int32, (N_RBF, 1), 0) + 1
             ).astype(jnp.float32)
    theta_t = jax.lax.dot_general(
        k_col, lengths * (math.pi / CUTOFF), (((1,), (0,)), ((), ())),
        precision=lax.Precision.HIGHEST)
    radial_t = jnp.sin(theta_t)

    x = unit[0:1]
    y = unit[1:2]
    z = unit[2:3]
    one = jnp.ones_like(x)
    es0 = gst[3:4]
    es1 = gst[4:5]
    er0 = grt[3:4]
    er1 = grt[4:5]
    enct = jnp.concatenate([es0 * er0, es0 * er1, es1 * er0, es1 * er1],
                           axis=0) * pref
    monos = [one, x, y, z, x * x, x * y, x * z, y * y, y * z, z * z]
    ae_t = jnp.concatenate([mono * enct for mono in monos], axis=0)

    wcat = jnp.concatenate([w_ref[0], w_ref[1], w_ref[2]], axis=1)
    w320 = jax.lax.dot_general(wcat, m1_ref[...], (((1,), (0,)), ((), ())),
                               precision=lax.Precision.HIGHEST)

    dot0 = functools.partial(jax.lax.dot_general,
                             dimension_numbers=(((0,), (0,)), ((), ())),
                             precision=lax.Precision.DEFAULT)

    def bf16_split(a):
        a_hi = a.astype(jnp.bfloat16).astype(jnp.float32)
        return a_hi, a - a_hi

    r_hi, r_lo = bf16_split(radial_t)
    w_hi, w_lo = bf16_split(w320)
    g1 = dot0(r_hi, w_hi) + dot0(r_hi, w_lo) + dot0(r_lo, w_hi)
    ae_hi, ae_lo = bf16_split(ae_t)
    gae = dot0(ae_hi, m23_ref[...]) + dot0(ae_lo, m23_ref[...])
    g = g1 * gae
    out_a[...] = g[:, 0:128]
    out_b[...] = g[:, 128:256]
    out_c[:, 0:64] = g[:, 256:320]


def _edge_expand(g, shifts_t, radial_transform_W):
    grid = N_EDGES // _EB
    nch = _EB // _CHUNK
    m1, m23 = _np_masks()
    slab = jax.ShapeDtypeStruct((N_EDGES, 128), jnp.float32)
    return pl.pallas_call(
        _edge_body,
        grid=(grid,),
        in_specs=[
            pl.BlockSpec((nch, 16, _CHUNK), lambda i: (i, 0, 0)),
            pl.BlockSpec((nch, 16, _CHUNK),
                         lambda i: (_N_CHUNKS // nch + i, 0, 0)),
            pl.BlockSpec((3, _EB), lambda i: (0, i)),
            pl.BlockSpec((3, 8, 8), lambda i: (0, 0, 0)),
            pl.BlockSpec((24, 320), lambda i: (0, 0)),
            pl.BlockSpec((40, 320), lambda i: (0, 0)),
        ],
        out_specs=[pl.BlockSpec((_EB, 128), lambda i: (i, 0))] * 3,
        out_shape=[slab, slab, slab],
    )(g, g, shifts_t, radial_transform_W, m1, m23)


def _scatter_body(ea_hbm, eb_hbm, ec_hbm, recv_hbm, zer_hbm, out_hbm,
                  idx_v, rows_v, acc):
    cid = lax.axis_index("c")
    tid = lax.axis_index("s")

    pltpu.sync_copy(zer_hbm, acc.at[pl.ds(tid * _STRIPE, _STRIPE), :])
    plsc.subcore_barrier()

    def step(k, carry):
        c = tid + 16 * k

        @pl.when(c < _N_CHUNKS)
        def _():
            rows = pl.ds(c * _CHUNK, _CHUNK)
            pltpu.sync_copy(recv_hbm.at[c], idx_v)
            @pl.when(cid == 0)
            def _():
                pltpu.sync_copy(ea_hbm.at[rows, :], rows_v.at[:, pl.ds(0, 128)])
                pltpu.sync_copy(eb_hbm.at[rows, pl.ds(0, 32)],
                                rows_v.at[:, pl.ds(128, 32)])

            @pl.when(cid == 1)
            def _():
                pltpu.sync_copy(eb_hbm.at[rows, pl.ds(32, 96)],
                                rows_v.at[:, pl.ds(0, 96)])
                pltpu.sync_copy(ec_hbm.at[rows, pl.ds(0, 64)],
                                rows_v.at[:, pl.ds(96, 64)])
            pltpu.sync_copy(rows_v, acc.at[idx_v], add=True)
        return carry

    lax.fori_loop(0, (_N_CHUNKS + 15) // 16, step, 0)
    plsc.subcore_barrier()
    pltpu.sync_copy(acc.at[pl.ds(tid * _STRIPE, _STRIPE), :],
                    out_hbm.at[cid, pl.ds(tid * _STRIPE, _STRIPE), :])


def _sc_scatter(ea, eb, ec, recv2d, zer):
    f = pl.kernel(
        _scatter_body,
        out_type=jax.ShapeDtypeStruct((2, N_NODES, 160), jnp.float32),
        mesh=plsc.VectorSubcoreMesh(core_axis_name="c", subcore_axis_name="s"),
        scratch_types=[
            pltpu.VMEM((_CHUNK,), jnp.int32),
            pltpu.VMEM((_CHUNK, 160), jnp.float32),
            pltpu.VMEM_SHARED((N_NODES, 160), jnp.float32),
        ],
        compiler_params=pltpu.CompilerParams(use_tc_tiling_on_sc=False),
    )
    return f(ea, eb, ec, recv2d, zer)


def _sym_body(nf_ref, out_ref):
    h0 = nf_ref[0]
    h1 = nf_ref[1]
    nu1 = h0[:, 0:32]
    nu21 = (h0[:, 32:64] * h0[:, 32:64]
            + h0[:, 64:96] * h0[:, 64:96]
            + h0[:, 96:128] * h0[:, 96:128])
    a4 = h0[:, 128:160]
    nu22 = _PREF2[0] * a4 * a4
    for j, pref in enumerate(_PREF2[1:]):
        blk = h1[:, 32 * j:32 * j + 32]
        nu22 = nu22 + pref * blk * blk
    pieces = []
    for s in range(8):
        pieces.append(nu1[:, 4 * s:4 * s + 4])
        pieces.append(nu21[:, 4 * s:4 * s + 4])
        pieces.append(nu22[:, 4 * s:4 * s + 4])
    out_ref[...] = jnp.concatenate(pieces, axis=1)


def _symmetrize(nfa):
    grid = N_NODES // _NB
    return pl.pallas_call(
        _sym_body,
        grid=(grid,),
        in_specs=[pl.BlockSpec((2, _NB, 160), lambda i: (0, i, 0))],
        out_specs=pl.BlockSpec((_NB, 96), lambda i: (i, 0)),
        out_shape=jax.ShapeDtypeStruct((N_NODES, 96), jnp.float32),
    )(nfa)


def kernel(positions, atomic_numbers, edge_index, shifts,
           node_embedding_W, radial_transform_W):
    n = positions.shape[0]
    z2d = atomic_numbers.reshape(n, 1).astype(jnp.int32)
    table = _build_table(positions, z2d, node_embedding_W)

    idx2d = edge_index.astype(jnp.int32).reshape(2 * _N_CHUNKS, _CHUNK)
    gathered = _sc_gather(table, idx2d)

    ea, eb, ec = _edge_expand(gathered, shifts.T, radial_transform_W)

    recv2d = edge_index[1].astype(jnp.int32).reshape(_N_CHUNKS, _CHUNK)
    zer = jnp.zeros((_STRIPE, 160), jnp.float32)
    nfa = _sc_scatter(ea, eb, ec, recv2d, zer)

    out96 = _symmetrize(nfa)
    return out96.reshape(n, 8, 3, 4)


# speedup vs baseline: 29.8314x; 1.0817x over previous
import functools
import math

import jax
import jax.numpy as jnp
import numpy as np
from jax import lax
from jax.experimental import pallas as pl
from jax.experimental.pallas import tpu as pltpu
from jax.experimental.pallas import tpu_sc as plsc

N_NODES = 10000
N_EDGES = 160000
N_RBF = 8
CUTOFF = 5.5
ZS_VALS = (1, 6, 7, 8)

_LXLYLZ = [(0, 0, 0),
           (1, 0, 0), (0, 1, 0), (0, 0, 1),
           (2, 0, 0), (1, 1, 0), (1, 0, 1), (0, 2, 0), (0, 1, 1), (0, 0, 2)]
_L_OF = [0, 1, 1, 1, 2, 2, 2, 2, 2, 2]
_PREF2 = [1.0, 2.0, 2.0, 1.0, 2.0, 1.0]

_CHUNK = 128
_N_CHUNKS = N_EDGES // _CHUNK
_STRIPE = N_NODES // 16
_EB = 640
_NB = 400


def _table_body(pos_ref, z_ref, w_ref, out_ref):
    z = z_ref[...]
    col = lax.broadcasted_iota(jnp.int32, (1, 4), 1)
    zs = ((col == 0) * ZS_VALS[0] + (col == 1) * ZS_VALS[1]
          + (col == 2) * ZS_VALS[2] + (col == 3) * ZS_VALS[3])
    one_hot = (z == zs).astype(jnp.float32)
    emb = jax.lax.dot_general(one_hot, w_ref[...], (((1,), (0,)), ((), ())),
                              precision=lax.Precision.HIGHEST)
    out_ref[:, 0:3] = pos_ref[...]
    out_ref[:, 3:5] = emb
    out_ref[:, 5:16] = jnp.zeros((z.shape[0], 11), jnp.float32)


def _build_table(positions, z2d, node_embedding_W):
    return pl.pallas_call(
        _table_body,
        out_shape=jax.ShapeDtypeStruct((N_NODES, 16), jnp.float32),
    )(positions, z2d, node_embedding_W)


def _gather_body(table_hbm, idx_hbm, out_hbm,
                 idx_v0, idx_v1, rows_v0, rows_v1, trows_v0, trows_v1,
                 gsem0, gsem1, osem0, osem1):
    cid = lax.axis_index("c")
    sid = lax.axis_index("s")
    wid = sid * 2 + cid
    n_rows = 2 * _N_CHUNKS
    n_k = (n_rows + 31) // 32
    lane = lax.iota(jnp.int32, 16)
    idx_v = (idx_v0, idx_v1)
    rows_v = (rows_v0, rows_v1)
    trows_v = (trows_v0, trows_v1)
    gsem = (gsem0, gsem1)
    osem = (osem0, osem1)

    def start_gather(k, b):
        @pl.when(k < n_k)
        def _():
            cc = wid + 32 * k

            @pl.when(cc < n_rows)
            def _():
                pltpu.sync_copy(idx_hbm.at[cc], idx_v[b])
                pltpu.async_copy(table_hbm.at[idx_v[b]], rows_v[b], gsem[b])

    def wait_gather(k, b):
        @pl.when(jnp.logical_and(k < n_k, wid + 32 * k < n_rows))
        def _():
            pltpu.make_async_copy(table_hbm.at[idx_v[b]], rows_v[b],
                                  gsem[b]).wait()

    def wait_out(k, b):
        @pl.when(jnp.logical_and(k >= 0, wid + 32 * k < n_rows))
        def _():
            pltpu.make_async_copy(trows_v[b], out_hbm.at[0], osem[b]).wait()

    def transpose_and_store(k, b):
        cc = wid + 32 * k

        @pl.when(jnp.logical_and(k < n_k, cc < n_rows))
        def _():
            for t in range(16):
                t_idx = jnp.full((16,), t, jnp.int32)
                for gi in range(8):
                    vals = plsc.load_gather(rows_v[b],
                                            [lane + 16 * gi, t_idx])
                    trows_v[b][t, pl.ds(16 * gi, 16)] = vals
            pltpu.async_copy(trows_v[b], out_hbm.at[cc], osem[b])

    start_gather(0, 0)
    start_gather(1, 1)

    def step(j, carry):
        for b in range(2):
            k = 2 * j + b
            wait_gather(k, b)
            wait_out(k - 2, b)
            transpose_and_store(k, b)
            start_gather(k + 2, b)
        return carry

    n_j = (n_k + 1) // 2
    lax.fori_loop(0, n_j, step, 0)
    wait_out(2 * n_j - 2, 0)
    wait_out(2 * n_j - 1, 1)


def _sc_gather(table, idx2d):
    f = pl.kernel(
        _gather_body,
        out_type=jax.ShapeDtypeStruct((2 * _N_CHUNKS, 16, _CHUNK),
                                      jnp.float32),
        mesh=plsc.VectorSubcoreMesh(core_axis_name="c", subcore_axis_name="s"),
        scratch_types=[
            pltpu.VMEM((_CHUNK,), jnp.int32),
            pltpu.VMEM((_CHUNK,), jnp.int32),
            pltpu.VMEM((_CHUNK, 16), jnp.float32),
            pltpu.VMEM((_CHUNK, 16), jnp.float32),
            pltpu.VMEM((16, _CHUNK), jnp.float32),
            pltpu.VMEM((16, _CHUNK), jnp.float32),
            pltpu.SemaphoreType.DMA,
            pltpu.SemaphoreType.DMA,
            pltpu.SemaphoreType.DMA,
            pltpu.SemaphoreType.DMA,
        ],
        compiler_params=pltpu.CompilerParams(use_tc_tiling_on_sc=False,
                                             needs_layout_passes=False),
    )
    return f(table, idx2d)


def _np_masks():
    col = np.arange(320)
    a_m = col // 32
    s_m = (col % 32) // 4
    c_m = col % 4
    l_m = (a_m >= 1).astype(np.int32) + (a_m >= 4).astype(np.int32)
    m1 = (np.arange(24)[:, None] == (l_m * 8 + s_m)[None, :]
          ).astype(np.float32)
    m23 = (np.arange(40)[:, None] == (a_m * 4 + c_m)[None, :]
           ).astype(np.float32)
    return m1, m23


def _edge_body(gs_ref, gr_ref, shifts_ref, w_ref, m1_ref, m23_ref,
               out_a, out_b, out_c):
    nch = _EB // _CHUNK
    gst = jnp.concatenate([gs_ref[i] for i in range(nch)], axis=1)
    grt = jnp.concatenate([gr_ref[i] for i in range(nch)], axis=1)
    vec = grt[0:3] - gst[0:3] + shifts_ref[...]
    d2 = jnp.sum(vec * vec, axis=0, keepdims=True)
    lengths = jnp.sqrt(d2 + 1e-12)
    inv_len = 1.0 / lengths
    unit = vec * inv_len

    r = lengths * (1.0 / CUTOFF)
    r2 = r * r
    r6 = r2 * r2 * r2
    r7 = r6 * r
    r8 = r7 * r
    fc = 1.0 - 28.0 * r6 + 48.0 * r7 - 21.0 * r8
    fc = jnp.where(lengths < CUTOFF, fc, 0.0)
    pref = fc * (math.sqrt(2.0 / CUTOFF)) * inv_len

    k_col = (lax.broadcasted_iota(jnp.int32, (N_RBF, 1), 0) + 1
             ).astype(jnp.float32)
    theta_t = jax.lax.dot_general(
        k_col, lengths * (math.pi / CUTOFF), (((1,), (0,)), ((), ())),
        precision=lax.Precision.HIGHEST)
    radial_t = jnp.sin(theta_t)

    x = unit[0:1]
    y = unit[1:2]
    z = unit[2:3]
    one = jnp.ones_like(x)
    es0 = gst[3:4]
    es1 = gst[4:5]
    er0 = grt[3:4]
    er1 = grt[4:5]
    enct = jnp.concatenate([es0 * er0, es0 * er1, es1 * er0, es1 * er1],
                           axis=0) * pref
    monos = [one, x, y, z, x * x, x * y, x * z, y * y, y * z, z * z]
    ae_t = jnp.concatenate([mono * enct for mono in monos], axis=0)

    wcat = jnp.concatenate([w_ref[0], w_ref[1], w_ref[2]], axis=1)
    w320 = jax.lax.dot_general(wcat, m1_ref[...], (((1,), (0,)), ((), ())),
                               precision=lax.Precision.HIGHEST)

    dot0 = functools.partial(jax.lax.dot_general,
                             dimension_numbers=(((0,), (0,)), ((), ())),
                             precision=lax.Precision.DEFAULT)

    def bf16_split(a):
        a_hi = a.astype(jnp.bfloat16).astype(jnp.float32)
        return a_hi, a - a_hi

    r_hi, r_lo = bf16_split(radial_t)
    w_hi, w_lo = bf16_split(w320)
    g1 = dot0(r_hi, w_hi) + dot0(r_hi, w_lo) + dot0(r_lo, w_hi)
    ae_hi, ae_lo = bf16_split(ae_t)
    gae = dot0(ae_hi, m23_ref[...]) + dot0(ae_lo, m23_ref[...])
    g = g1 * gae
    out_a[...] = g[:, 0:128]
    out_b[...] = g[:, 128:256]
    out_c[:, 0:64] = g[:, 256:320]


def _edge_expand(g, shifts_t, radial_transform_W):
    grid = N_EDGES // _EB
    nch = _EB // _CHUNK
    m1, m23 = _np_masks()
    slab = jax.ShapeDtypeStruct((N_EDGES, 128), jnp.float32)
    return pl.pallas_call(
        _edge_body,
        grid=(grid,),
        in_specs=[
            pl.BlockSpec((nch, 16, _CHUNK), lambda i: (i, 0, 0)),
            pl.BlockSpec((nch, 16, _CHUNK),
                         lambda i: (_N_CHUNKS // nch + i, 0, 0)),
            pl.BlockSpec((3, _EB), lambda i: (0, i)),
            pl.BlockSpec((3, 8, 8), lambda i: (0, 0, 0)),
            pl.BlockSpec((24, 320), lambda i: (0, 0)),
            pl.BlockSpec((40, 320), lambda i: (0, 0)),
        ],
        out_specs=[pl.BlockSpec((_EB, 128), lambda i: (i, 0))] * 3,
        out_shape=[slab, slab, slab],
    )(g, g, shifts_t, radial_transform_W, m1, m23)


def _scatter_body(ea_hbm, eb_hbm, ec_hbm, recv_hbm, zer_hbm, out_hbm,
                  idx_v0, idx_v1, rows_v0, rows_v1, acc,
                  rsem0, rsem1, ssem0, ssem1):
    cid = lax.axis_index("c")
    tid = lax.axis_index("s")
    n_k = (_N_CHUNKS + 15) // 16
    idx_v = (idx_v0, idx_v1)
    rows_v = (rows_v0, rows_v1)
    rsem = (rsem0, rsem1)
    ssem = (ssem0, ssem1)

    pltpu.sync_copy(zer_hbm, acc.at[pl.ds(tid * _STRIPE, _STRIPE), :])
    plsc.subcore_barrier()

    def live(k):
        return jnp.logical_and(k >= 0, tid + 16 * k < _N_CHUNKS)

    def do_read(k, b):
        @pl.when(live(k))
        def _():
            c = tid + 16 * k
            rows = pl.ds(c * _CHUNK, _CHUNK)
            pltpu.sync_copy(recv_hbm.at[c], idx_v[b])
            @pl.when(cid == 0)
            def _():
                pltpu.sync_copy(ea_hbm.at[rows, :],
                                rows_v[b].at[:, pl.ds(0, 128)])
                pltpu.sync_copy(eb_hbm.at[rows, pl.ds(0, 32)],
                                rows_v[b].at[:, pl.ds(128, 32)])

            @pl.when(cid == 1)
            def _():
                pltpu.sync_copy(eb_hbm.at[rows, pl.ds(32, 96)],
                                rows_v[b].at[:, pl.ds(0, 96)])
                pltpu.sync_copy(ec_hbm.at[rows, pl.ds(0, 64)],
                                rows_v[b].at[:, pl.ds(96, 64)])

    def start_scatter(k, b):
        @pl.when(live(k))
        def _():
            pltpu.async_copy(rows_v[b], acc.at[idx_v[b]], ssem[b], add=True)

    def wait_scatter(k, b):
        @pl.when(live(k))
        def _():
            pltpu.make_async_copy(rows_v[b], acc.at[idx_v[b]],
                                  ssem[b]).wait()

    def step(k, carry):
        do_read(k, 0)

        @pl.when(live(k))
        def _():
            pltpu.sync_copy(rows_v[0], acc.at[idx_v[0]], add=True)
        return carry

    lax.fori_loop(0, n_k, step, 0)
    plsc.subcore_barrier()
    pltpu.sync_copy(acc.at[pl.ds(tid * _STRIPE, _STRIPE), :],
                    out_hbm.at[cid, pl.ds(tid * _STRIPE, _STRIPE), :])


def _sc_scatter(ea, eb, ec, recv2d, zer):
    f = pl.kernel(
        _scatter_body,
        out_type=jax.ShapeDtypeStruct((2, N_NODES, 160), jnp.float32),
        mesh=plsc.VectorSubcoreMesh(core_axis_name="c", subcore_axis_name="s"),
        scratch_types=[
            pltpu.VMEM((_CHUNK,), jnp.int32),
            pltpu.VMEM((_CHUNK,), jnp.int32),
            pltpu.VMEM((_CHUNK, 160), jnp.float32),
            pltpu.VMEM((_CHUNK, 160), jnp.float32),
            pltpu.VMEM_SHARED((N_NODES, 160), jnp.float32),
            pltpu.SemaphoreType.DMA,
            pltpu.SemaphoreType.DMA,
            pltpu.SemaphoreType.DMA,
            pltpu.SemaphoreType.DMA,
        ],
        compiler_params=pltpu.CompilerParams(use_tc_tiling_on_sc=False),
    )
    return f(ea, eb, ec, recv2d, zer)


def _sym_body(nf_ref, out_ref):
    h0 = nf_ref[0]
    h1 = nf_ref[1]
    nu1 = h0[:, 0:32]
    nu21 = (h0[:, 32:64] * h0[:, 32:64]
            + h0[:, 64:96] * h0[:, 64:96]
            + h0[:, 96:128] * h0[:, 96:128])
    a4 = h0[:, 128:160]
    nu22 = _PREF2[0] * a4 * a4
    for j, pref in enumerate(_PREF2[1:]):
        blk = h1[:, 32 * j:32 * j + 32]
        nu22 = nu22 + pref * blk * blk
    pieces = []
    for s in range(8):
        pieces.append(nu1[:, 4 * s:4 * s + 4])
        pieces.append(nu21[:, 4 * s:4 * s + 4])
        pieces.append(nu22[:, 4 * s:4 * s + 4])
    out_ref[...] = jnp.concatenate(pieces, axis=1)


def _symmetrize(nfa):
    grid = N_NODES // _NB
    return pl.pallas_call(
        _sym_body,
        grid=(grid,),
        in_specs=[pl.BlockSpec((2, _NB, 160), lambda i: (0, i, 0))],
        out_specs=pl.BlockSpec((_NB, 96), lambda i: (i, 0)),
        out_shape=jax.ShapeDtypeStruct((N_NODES, 96), jnp.float32),
    )(nfa)


def kernel(positions, atomic_numbers, edge_index, shifts,
           node_embedding_W, radial_transform_W):
    n = positions.shape[0]
    z2d = atomic_numbers.reshape(n, 1).astype(jnp.int32)
    table = _build_table(positions, z2d, node_embedding_W)

    idx2d = edge_index.astype(jnp.int32).reshape(2 * _N_CHUNKS, _CHUNK)
    gathered = _sc_gather(table, idx2d)

    ea, eb, ec = _edge_expand(gathered, shifts.T, radial_transform_W)

    recv2d = edge_index[1].astype(jnp.int32).reshape(_N_CHUNKS, _CHUNK)
    zer = jnp.zeros((_STRIPE, 160), jnp.float32)
    nfa = _sc_scatter(ea, eb, ec, recv2d, zer)

    out96 = _symmetrize(nfa)
    return out96.reshape(n, 8, 3, 4)


# speedup vs baseline: 32.3397x; 1.0841x over previous
import functools
import math

import jax
import jax.numpy as jnp
import numpy as np
from jax import lax
from jax.experimental import pallas as pl
from jax.experimental.pallas import tpu as pltpu
from jax.experimental.pallas import tpu_sc as plsc

N_NODES = 10000
N_EDGES = 160000
N_RBF = 8
CUTOFF = 5.5
ZS_VALS = (1, 6, 7, 8)

_LXLYLZ = [(0, 0, 0),
           (1, 0, 0), (0, 1, 0), (0, 0, 1),
           (2, 0, 0), (1, 1, 0), (1, 0, 1), (0, 2, 0), (0, 1, 1), (0, 0, 2)]
_L_OF = [0, 1, 1, 1, 2, 2, 2, 2, 2, 2]
_PREF2 = [1.0, 2.0, 2.0, 1.0, 2.0, 1.0]

_CHUNK = 128
_N_CHUNKS = N_EDGES // _CHUNK
_STRIPE = N_NODES // 16
_EB = 1280
_NB = 400


def _table_body(pos_ref, z_ref, w_ref, out_ref):
    z = z_ref[...]
    col = lax.broadcasted_iota(jnp.int32, (1, 4), 1)
    zs = ((col == 0) * ZS_VALS[0] + (col == 1) * ZS_VALS[1]
          + (col == 2) * ZS_VALS[2] + (col == 3) * ZS_VALS[3])
    one_hot = (z == zs).astype(jnp.float32)
    emb = jax.lax.dot_general(one_hot, w_ref[...], (((1,), (0,)), ((), ())),
                              precision=lax.Precision.HIGHEST)
    out_ref[:, 0:3] = pos_ref[...]
    out_ref[:, 3:5] = emb
    out_ref[:, 5:16] = jnp.zeros((z.shape[0], 11), jnp.float32)


def _build_table(positions, z2d, node_embedding_W):
    return pl.pallas_call(
        _table_body,
        out_shape=jax.ShapeDtypeStruct((N_NODES, 16), jnp.float32),
    )(positions, z2d, node_embedding_W)


def _gather_body(table_hbm, idx_hbm, out_hbm,
                 idx_v0, idx_v1, rows_v0, rows_v1, trows_v0, trows_v1,
                 gsem0, gsem1, osem0, osem1):
    cid = lax.axis_index("c")
    sid = lax.axis_index("s")
    wid = sid * 2 + cid
    n_rows = 2 * _N_CHUNKS
    n_k = (n_rows + 31) // 32
    lane = lax.iota(jnp.int32, 16)
    idx_v = (idx_v0, idx_v1)
    rows_v = (rows_v0, rows_v1)
    trows_v = (trows_v0, trows_v1)
    gsem = (gsem0, gsem1)
    osem = (osem0, osem1)

    def start_gather(k, b):
        @pl.when(k < n_k)
        def _():
            cc = wid + 32 * k

            @pl.when(cc < n_rows)
            def _():
                pltpu.sync_copy(idx_hbm.at[cc], idx_v[b])
                pltpu.async_copy(table_hbm.at[idx_v[b]], rows_v[b], gsem[b])

    def wait_gather(k, b):
        @pl.when(jnp.logical_and(k < n_k, wid + 32 * k < n_rows))
        def _():
            pltpu.make_async_copy(table_hbm.at[idx_v[b]], rows_v[b],
                                  gsem[b]).wait()

    def wait_out(k, b):
        @pl.when(jnp.logical_and(k >= 0, wid + 32 * k < n_rows))
        def _():
            pltpu.make_async_copy(trows_v[b], out_hbm.at[0], osem[b]).wait()

    def transpose_and_store(k, b):
        cc = wid + 32 * k

        @pl.when(jnp.logical_and(k < n_k, cc < n_rows))
        def _():
            for t in range(16):
                t_idx = jnp.full((16,), t, jnp.int32)
                for gi in range(8):
                    vals = plsc.load_gather(rows_v[b],
                                            [lane + 16 * gi, t_idx])
                    trows_v[b][t, pl.ds(16 * gi, 16)] = vals
            pltpu.async_copy(trows_v[b], out_hbm.at[cc], osem[b])

    start_gather(0, 0)
    start_gather(1, 1)

    def step(j, carry):
        for b in range(2):
            k = 2 * j + b
            wait_gather(k, b)
            wait_out(k - 2, b)
            transpose_and_store(k, b)
            start_gather(k + 2, b)
        return carry

    n_j = (n_k + 1) // 2
    lax.fori_loop(0, n_j, step, 0)
    wait_out(2 * n_j - 2, 0)
    wait_out(2 * n_j - 1, 1)


def _sc_gather(table, idx2d):
    f = pl.kernel(
        _gather_body,
        out_type=jax.ShapeDtypeStruct((2 * _N_CHUNKS, 16, _CHUNK),
                                      jnp.float32),
        mesh=plsc.VectorSubcoreMesh(core_axis_name="c", subcore_axis_name="s"),
        scratch_types=[
            pltpu.VMEM((_CHUNK,), jnp.int32),
            pltpu.VMEM((_CHUNK,), jnp.int32),
            pltpu.VMEM((_CHUNK, 16), jnp.float32),
            pltpu.VMEM((_CHUNK, 16), jnp.float32),
            pltpu.VMEM((16, _CHUNK), jnp.float32),
            pltpu.VMEM((16, _CHUNK), jnp.float32),
            pltpu.SemaphoreType.DMA,
            pltpu.SemaphoreType.DMA,
            pltpu.SemaphoreType.DMA,
            pltpu.SemaphoreType.DMA,
        ],
        compiler_params=pltpu.CompilerParams(use_tc_tiling_on_sc=False,
                                             needs_layout_passes=False),
    )
    return f(table, idx2d)


def _np_masks():
    col = np.arange(320)
    a_m = col // 32
    s_m = (col % 32) // 4
    c_m = col % 4
    l_m = (a_m >= 1).astype(np.int32) + (a_m >= 4).astype(np.int32)
    m1 = (np.arange(24)[:, None] == (l_m * 8 + s_m)[None, :]
          ).astype(np.float32)
    m23 = (np.arange(40)[:, None] == (a_m * 4 + c_m)[None, :]
           ).astype(np.float32)
    return m1, m23


def _edge_body(gs_ref, gr_ref, shifts_ref, w_ref, m1_ref, m23_ref,
               out_a, out_b, out_c):
    nch = _EB // _CHUNK
    gst = jnp.concatenate([gs_ref[i] for i in range(nch)], axis=1)
    grt = jnp.concatenate([gr_ref[i] for i in range(nch)], axis=1)
    vec = grt[0:3] - gst[0:3] + shifts_ref[...]
    d2 = jnp.sum(vec * vec, axis=0, keepdims=True)
    lengths = jnp.sqrt(d2 + 1e-12)
    inv_len = 1.0 / lengths
    unit = vec * inv_len

    r = lengths * (1.0 / CUTOFF)
    r2 = r * r
    r6 = r2 * r2 * r2
    r7 = r6 * r
    r8 = r7 * r
    fc = 1.0 - 28.0 * r6 + 48.0 * r7 - 21.0 * r8
    fc = jnp.where(lengths < CUTOFF, fc, 0.0)
    pref = fc * (math.sqrt(2.0 / CUTOFF)) * inv_len

    k_col = (lax.broadcasted_iota(jnp.int32, (N_RBF, 1), 0) + 1
             ).astype(jnp.float32)
    theta_t = jax.lax.dot_general(
        k_col, lengths * (math.pi / CUTOFF), (((1,), (0,)), ((), ())),
        precision=lax.Precision.HIGHEST)
    radial_t = jnp.sin(theta_t)

    x = unit[0:1]
    y = unit[1:2]
    z = unit[2:3]
    one = jnp.ones_like(x)
    es0 = gst[3:4]
    es1 = gst[4:5]
    er0 = grt[3:4]
    er1 = grt[4:5]
    enct = jnp.concatenate([es0 * er0, es0 * er1, es1 * er0, es1 * er1],
                           axis=0) * pref
    monos = [one, x, y, z, x * x, x * y, x * z, y * y, y * z, z * z]
    ae_t = jnp.concatenate([mono * enct for mono in monos], axis=0)

    wcat = jnp.concatenate([w_ref[0], w_ref[1], w_ref[2]], axis=1)
    w320 = jax.lax.dot_general(wcat, m1_ref[...], (((1,), (0,)), ((), ())),
                               precision=lax.Precision.HIGHEST)

    dot0 = functools.partial(jax.lax.dot_general,
                             dimension_numbers=(((0,), (0,)), ((), ())),
                             precision=lax.Precision.DEFAULT)

    def bf16_split(a):
        a_hi = a.astype(jnp.bfloat16).astype(jnp.float32)
        return a_hi, a - a_hi

    r_hi, r_lo = bf16_split(radial_t)
    w_hi, w_lo = bf16_split(w320)
    g1 = dot0(r_hi, w_hi) + dot0(r_hi, w_lo) + dot0(r_lo, w_hi)
    ae_hi, ae_lo = bf16_split(ae_t)
    gae = dot0(ae_hi, m23_ref[...]) + dot0(ae_lo, m23_ref[...])
    g = g1 * gae
    out_a[...] = g[:, 0:128]
    out_b[...] = g[:, 128:256]
    out_c[:, 0:64] = g[:, 256:320]


def _edge_expand(g, shifts_t, radial_transform_W):
    grid = N_EDGES // _EB
    nch = _EB // _CHUNK
    m1, m23 = _np_masks()
    slab = jax.ShapeDtypeStruct((N_EDGES, 128), jnp.float32)
    return pl.pallas_call(
        _edge_body,
        grid=(grid,),
        in_specs=[
            pl.BlockSpec((nch, 16, _CHUNK), lambda i: (i, 0, 0)),
            pl.BlockSpec((nch, 16, _CHUNK),
                         lambda i: (_N_CHUNKS // nch + i, 0, 0)),
            pl.BlockSpec((3, _EB), lambda i: (0, i)),
            pl.BlockSpec((3, 8, 8), lambda i: (0, 0, 0)),
            pl.BlockSpec((24, 320), lambda i: (0, 0)),
            pl.BlockSpec((40, 320), lambda i: (0, 0)),
        ],
        out_specs=[pl.BlockSpec((_EB, 128), lambda i: (i, 0))] * 3,
        out_shape=[slab, slab, slab],
    )(g, g, shifts_t, radial_transform_W, m1, m23)


def _scatter_body(ea_hbm, eb_hbm, ec_hbm, recv_hbm, zer_hbm, out_hbm,
                  idx_v0, idx_v1, rows_v0, rows_v1, acc,
                  rsem0, rsem1, ssem0, ssem1):
    cid = lax.axis_index("c")
    tid = lax.axis_index("s")
    n_k = (_N_CHUNKS + 15) // 16
    idx_v = (idx_v0, idx_v1)
    rows_v = (rows_v0, rows_v1)
    rsem = (rsem0, rsem1)
    ssem = (ssem0, ssem1)

    pltpu.sync_copy(zer_hbm, acc.at[pl.ds(tid * _STRIPE, _STRIPE), :])
    plsc.subcore_barrier()

    def live(k):
        return jnp.logical_and(k >= 0, tid + 16 * k < _N_CHUNKS)

    def do_read(k, b):
        @pl.when(live(k))
        def _():
            c = tid + 16 * k
            rows = pl.ds(c * _CHUNK, _CHUNK)
            pltpu.sync_copy(recv_hbm.at[c], idx_v[b])
            @pl.when(cid == 0)
            def _():
                pltpu.sync_copy(ea_hbm.at[rows, :],
                                rows_v[b].at[:, pl.ds(0, 128)])
                pltpu.sync_copy(eb_hbm.at[rows, pl.ds(0, 32)],
                                rows_v[b].at[:, pl.ds(128, 32)])

            @pl.when(cid == 1)
            def _():
                pltpu.sync_copy(eb_hbm.at[rows, pl.ds(32, 96)],
                                rows_v[b].at[:, pl.ds(0, 96)])
                pltpu.sync_copy(ec_hbm.at[rows, pl.ds(0, 64)],
                                rows_v[b].at[:, pl.ds(96, 64)])

    def start_scatter(k, b):
        @pl.when(live(k))
        def _():
            pltpu.async_copy(rows_v[b], acc.at[idx_v[b]], ssem[b], add=True)

    def wait_scatter(k, b):
        @pl.when(live(k))
        def _():
            pltpu.make_async_copy(rows_v[b], acc.at[idx_v[b]],
                                  ssem[b]).wait()

    def step(k, carry):
        do_read(k, 0)

        @pl.when(live(k))
        def _():
            pltpu.sync_copy(rows_v[0], acc.at[idx_v[0]], add=True)
        return carry

    lax.fori_loop(0, n_k, step, 0)
    plsc.subcore_barrier()
    pltpu.sync_copy(acc.at[pl.ds(tid * _STRIPE, _STRIPE), :],
                    out_hbm.at[cid, pl.ds(tid * _STRIPE, _STRIPE), :])


def _sc_scatter(ea, eb, ec, recv2d, zer):
    f = pl.kernel(
        _scatter_body,
        out_type=jax.ShapeDtypeStruct((2, N_NODES, 160), jnp.float32),
        mesh=plsc.VectorSubcoreMesh(core_axis_name="c", subcore_axis_name="s"),
        scratch_types=[
            pltpu.VMEM((_CHUNK,), jnp.int32),
            pltpu.VMEM((_CHUNK,), jnp.int32),
            pltpu.VMEM((_CHUNK, 160), jnp.float32),
            pltpu.VMEM((_CHUNK, 160), jnp.float32),
            pltpu.VMEM_SHARED((N_NODES, 160), jnp.float32),
            pltpu.SemaphoreType.DMA,
            pltpu.SemaphoreType.DMA,
            pltpu.SemaphoreType.DMA,
            pltpu.SemaphoreType.DMA,
        ],
        compiler_params=pltpu.CompilerParams(use_tc_tiling_on_sc=False),
    )
    return f(ea, eb, ec, recv2d, zer)


def _sym_body(nf_ref, out_ref):
    h0 = nf_ref[0]
    h1 = nf_ref[1]
    nu1 = h0[:, 0:32]
    nu21 = (h0[:, 32:64] * h0[:, 32:64]
            + h0[:, 64:96] * h0[:, 64:96]
            + h0[:, 96:128] * h0[:, 96:128])
    a4 = h0[:, 128:160]
    nu22 = _PREF2[0] * a4 * a4
    for j, pref in enumerate(_PREF2[1:]):
        blk = h1[:, 32 * j:32 * j + 32]
        nu22 = nu22 + pref * blk * blk
    pieces = []
    for s in range(8):
        pieces.append(nu1[:, 4 * s:4 * s + 4])
        pieces.append(nu21[:, 4 * s:4 * s + 4])
        pieces.append(nu22[:, 4 * s:4 * s + 4])
    out_ref[...] = jnp.concatenate(pieces, axis=1)


def _symmetrize(nfa):
    grid = N_NODES // _NB
    return pl.pallas_call(
        _sym_body,
        grid=(grid,),
        in_specs=[pl.BlockSpec((2, _NB, 160), lambda i: (0, i, 0))],
        out_specs=pl.BlockSpec((_NB, 96), lambda i: (i, 0)),
        out_shape=jax.ShapeDtypeStruct((N_NODES, 96), jnp.float32),
    )(nfa)


def kernel(positions, atomic_numbers, edge_index, shifts,
           node_embedding_W, radial_transform_W):
    n = positions.shape[0]
    z2d = atomic_numbers.reshape(n, 1).astype(jnp.int32)
    table = _build_table(positions, z2d, node_embedding_W)

    idx2d = edge_index.astype(jnp.int32).reshape(2 * _N_CHUNKS, _CHUNK)
    gathered = _sc_gather(table, idx2d)

    ea, eb, ec = _edge_expand(gathered, shifts.T, radial_transform_W)

    recv2d = edge_index[1].astype(jnp.int32).reshape(_N_CHUNKS, _CHUNK)
    zer = jnp.zeros((_STRIPE, 160), jnp.float32)
    nfa = _sc_scatter(ea, eb, ec, recv2d, zer)

    out96 = _symmetrize(nfa)
    return out96.reshape(n, 8, 3, 4)


# speedup vs baseline: 32.3423x; 1.0001x over previous
import functools
import math

import jax
import jax.numpy as jnp
import numpy as np
from jax import lax
from jax.experimental import pallas as pl
from jax.experimental.pallas import tpu as pltpu
from jax.experimental.pallas import tpu_sc as plsc

N_NODES = 10000
N_EDGES = 160000
N_RBF = 8
CUTOFF = 5.5
ZS_VALS = (1, 6, 7, 8)

_LXLYLZ = [(0, 0, 0),
           (1, 0, 0), (0, 1, 0), (0, 0, 1),
           (2, 0, 0), (1, 1, 0), (1, 0, 1), (0, 2, 0), (0, 1, 1), (0, 0, 2)]
_L_OF = [0, 1, 1, 1, 2, 2, 2, 2, 2, 2]
_PREF2 = [1.0, 2.0, 2.0, 1.0, 2.0, 1.0]

_CHUNK = 128
_N_CHUNKS = N_EDGES // _CHUNK
_STRIPE = N_NODES // 16
_EB = 1280
_NB = 400


def _table_body(pos_ref, z_ref, w_ref, out_ref):
    z = z_ref[...]
    col = lax.broadcasted_iota(jnp.int32, (1, 4), 1)
    zs = ((col == 0) * ZS_VALS[0] + (col == 1) * ZS_VALS[1]
          + (col == 2) * ZS_VALS[2] + (col == 3) * ZS_VALS[3])
    one_hot = (z == zs).astype(jnp.float32)
    emb = jax.lax.dot_general(one_hot, w_ref[...], (((1,), (0,)), ((), ())),
                              precision=lax.Precision.HIGHEST)
    out_ref[:, 0:3] = pos_ref[...]
    out_ref[:, 3:5] = emb
    out_ref[:, 5:16] = jnp.zeros((z.shape[0], 11), jnp.float32)


def _build_table(positions, z2d, node_embedding_W):
    return pl.pallas_call(
        _table_body,
        out_shape=jax.ShapeDtypeStruct((N_NODES, 16), jnp.float32),
    )(positions, z2d, node_embedding_W)


def _gather_body(table_hbm, idx_hbm, out_hbm,
                 idx_v0, idx_v1, rows_v0, rows_v1, trows_v0, trows_v1,
                 gsem0, gsem1, osem0, osem1):
    cid = lax.axis_index("c")
    sid = lax.axis_index("s")
    wid = sid * 2 + cid
    n_rows = 2 * _N_CHUNKS
    n_k = (n_rows + 31) // 32
    lane = lax.iota(jnp.int32, 16)
    idx_v = (idx_v0, idx_v1)
    rows_v = (rows_v0, rows_v1)
    trows_v = (trows_v0, trows_v1)
    gsem = (gsem0, gsem1)
    osem = (osem0, osem1)

    def start_gather(k, b):
        @pl.when(k < n_k)
        def _():
            cc = wid + 32 * k

            @pl.when(cc < n_rows)
            def _():
                pltpu.sync_copy(idx_hbm.at[cc], idx_v[b])
                pltpu.async_copy(table_hbm.at[idx_v[b]], rows_v[b], gsem[b])

    def wait_gather(k, b):
        @pl.when(jnp.logical_and(k < n_k, wid + 32 * k < n_rows))
        def _():
            pltpu.make_async_copy(table_hbm.at[idx_v[b]], rows_v[b],
                                  gsem[b]).wait()

    def wait_out(k, b):
        @pl.when(jnp.logical_and(k >= 0, wid + 32 * k < n_rows))
        def _():
            pltpu.make_async_copy(trows_v[b], out_hbm.at[0], osem[b]).wait()

    def transpose_and_store(k, b):
        cc = wid + 32 * k

        @pl.when(jnp.logical_and(k < n_k, cc < n_rows))
        def _():
            for t in range(16):
                t_idx = jnp.full((16,), t, jnp.int32)
                for gi in range(8):
                    vals = plsc.load_gather(rows_v[b],
                                            [lane + 16 * gi, t_idx])
                    trows_v[b][t, pl.ds(16 * gi, 16)] = vals
            pltpu.async_copy(trows_v[b], out_hbm.at[cc], osem[b])

    start_gather(0, 0)
    start_gather(1, 1)

    def step(j, carry):
        for b in range(2):
            k = 2 * j + b
            wait_gather(k, b)
            wait_out(k - 2, b)
            transpose_and_store(k, b)
            start_gather(k + 2, b)
        return carry

    n_j = (n_k + 1) // 2
    lax.fori_loop(0, n_j, step, 0)
    wait_out(2 * n_j - 2, 0)
    wait_out(2 * n_j - 1, 1)


def _sc_gather(table, idx2d):
    f = pl.kernel(
        _gather_body,
        out_type=jax.ShapeDtypeStruct((2 * _N_CHUNKS, 16, _CHUNK),
                                      jnp.float32),
        mesh=plsc.VectorSubcoreMesh(core_axis_name="c", subcore_axis_name="s"),
        scratch_types=[
            pltpu.VMEM((_CHUNK,), jnp.int32),
            pltpu.VMEM((_CHUNK,), jnp.int32),
            pltpu.VMEM((_CHUNK, 16), jnp.float32),
            pltpu.VMEM((_CHUNK, 16), jnp.float32),
            pltpu.VMEM((16, _CHUNK), jnp.float32),
            pltpu.VMEM((16, _CHUNK), jnp.float32),
            pltpu.SemaphoreType.DMA,
            pltpu.SemaphoreType.DMA,
            pltpu.SemaphoreType.DMA,
            pltpu.SemaphoreType.DMA,
        ],
        compiler_params=pltpu.CompilerParams(use_tc_tiling_on_sc=False,
                                             needs_layout_passes=False),
    )
    return f(table, idx2d)


def _np_masks():
    col = np.arange(320)
    a_m = col // 32
    s_m = (col % 32) // 4
    c_m = col % 4
    l_m = (a_m >= 1).astype(np.int32) + (a_m >= 4).astype(np.int32)
    m1 = (np.arange(24)[:, None] == (l_m * 8 + s_m)[None, :]
          ).astype(np.float32)
    m23 = (np.arange(40)[:, None] == (a_m * 4 + c_m)[None, :]
           ).astype(np.float32)
    return m1, m23


def _edge_body(gs_ref, gr_ref, shifts_ref, w_ref, m1_ref, m23_ref,
               out_a, out_b, out_c):
    nch = _EB // _CHUNK
    gst = jnp.concatenate([gs_ref[i] for i in range(nch)], axis=1)
    grt = jnp.concatenate([gr_ref[i] for i in range(nch)], axis=1)
    vec = grt[0:3] - gst[0:3] + shifts_ref[...]
    d2 = jnp.sum(vec * vec, axis=0, keepdims=True)
    lengths = jnp.sqrt(d2 + 1e-12)
    inv_len = 1.0 / lengths
    unit = vec * inv_len

    r = lengths * (1.0 / CUTOFF)
    r2 = r * r
    r6 = r2 * r2 * r2
    r7 = r6 * r
    r8 = r7 * r
    fc = 1.0 - 28.0 * r6 + 48.0 * r7 - 21.0 * r8
    fc = jnp.where(lengths < CUTOFF, fc, 0.0)
    pref = fc * (math.sqrt(2.0 / CUTOFF)) * inv_len

    k_col = (lax.broadcasted_iota(jnp.int32, (N_RBF, 1), 0) + 1
             ).astype(jnp.float32)
    theta_t = jax.lax.dot_general(
        k_col, lengths * (math.pi / CUTOFF), (((1,), (0,)), ((), ())),
        precision=lax.Precision.HIGHEST)
    radial_t = jnp.sin(theta_t)

    x = unit[0:1]
    y = unit[1:2]
    z = unit[2:3]
    one = jnp.ones_like(x)
    es0 = gst[3:4]
    es1 = gst[4:5]
    er0 = grt[3:4]
    er1 = grt[4:5]
    enct = jnp.concatenate([es0 * er0, es0 * er1, es1 * er0, es1 * er1],
                           axis=0) * pref
    monos = [one, x, y, z, x * x, x * y, x * z, y * y, y * z, z * z]
    ae_t = jnp.concatenate([mono * enct for mono in monos], axis=0)

    wcat = jnp.concatenate([w_ref[0], w_ref[1], w_ref[2]], axis=1)
    w320 = jax.lax.dot_general(wcat, m1_ref[...], (((1,), (0,)), ((), ())),
                               precision=lax.Precision.HIGHEST)

    dot0 = functools.partial(jax.lax.dot_general,
                             dimension_numbers=(((0,), (0,)), ((), ())),
                             precision=lax.Precision.DEFAULT)

    def bf16_split(a):
        a_hi = a.astype(jnp.bfloat16).astype(jnp.float32)
        return a_hi, a - a_hi

    r_hi, r_lo = bf16_split(radial_t)
    w_hi, w_lo = bf16_split(w320)
    g1 = dot0(r_hi, w_hi) + dot0(r_hi, w_lo) + dot0(r_lo, w_hi)
    ae_hi, ae_lo = bf16_split(ae_t)
    gae = dot0(ae_hi, m23_ref[...]) + dot0(ae_lo, m23_ref[...])
    g = g1 * gae
    out_a[...] = g[:, 0:128]
    out_b[...] = g[:, 128:256]
    out_c[:, 0:64] = g[:, 256:320]


def _edge_expand(g, shifts_t, radial_transform_W):
    grid = N_EDGES // _EB
    nch = _EB // _CHUNK
    m1, m23 = _np_masks()
    slab = jax.ShapeDtypeStruct((N_EDGES, 128), jnp.float32)
    return pl.pallas_call(
        _edge_body,
        grid=(grid,),
        in_specs=[
            pl.BlockSpec((nch, 16, _CHUNK), lambda i: (i, 0, 0)),
            pl.BlockSpec((nch, 16, _CHUNK),
                         lambda i: (_N_CHUNKS // nch + i, 0, 0)),
            pl.BlockSpec((3, _EB), lambda i: (0, i)),
            pl.BlockSpec((3, 8, 8), lambda i: (0, 0, 0)),
            pl.BlockSpec((24, 320), lambda i: (0, 0)),
            pl.BlockSpec((40, 320), lambda i: (0, 0)),
        ],
        out_specs=[pl.BlockSpec((_EB, 128), lambda i: (i, 0))] * 3,
        out_shape=[slab, slab, slab],
    )(g, g, shifts_t, radial_transform_W, m1, m23)


def _scatter_body(ea_hbm, eb_hbm, ec_hbm, recv_hbm, zer_hbm, out_hbm,
                  idx_v, rows_v, acc):
    cid = lax.axis_index("c")
    tid = lax.axis_index("s")
    n_k = (_N_CHUNKS + 15) // 16

    pltpu.sync_copy(zer_hbm, acc.at[pl.ds(tid * _STRIPE, _STRIPE), :])
    plsc.subcore_barrier()

    def step(k, carry):
        c = tid + 16 * k

        @pl.when(c < _N_CHUNKS)
        def _():
            rows = pl.ds(c * _CHUNK, _CHUNK)
            pltpu.sync_copy(recv_hbm.at[c], idx_v)
            @pl.when(cid == 0)
            def _():
                pltpu.sync_copy(ea_hbm.at[rows, :],
                                rows_v.at[:, pl.ds(0, 128)])
                pltpu.sync_copy(eb_hbm.at[rows, pl.ds(0, 32)],
                                rows_v.at[:, pl.ds(128, 32)])

            @pl.when(cid == 1)
            def _():
                pltpu.sync_copy(eb_hbm.at[rows, pl.ds(32, 96)],
                                rows_v.at[:, pl.ds(0, 96)])
                pltpu.sync_copy(ec_hbm.at[rows, pl.ds(0, 64)],
                                rows_v.at[:, pl.ds(96, 64)])
            pltpu.sync_copy(rows_v, acc.at[idx_v], add=True)
        return carry

    lax.fori_loop(0, n_k, step, 0)
    plsc.subcore_barrier()
    pltpu.sync_copy(acc.at[pl.ds(tid * _STRIPE, _STRIPE), :],
                    out_hbm.at[cid, pl.ds(tid * _STRIPE, _STRIPE), :])


def _sc_scatter(ea, eb, ec, recv2d, zer):
    f = pl.kernel(
        _scatter_body,
        out_type=jax.ShapeDtypeStruct((2, N_NODES, 160), jnp.float32),
        mesh=plsc.VectorSubcoreMesh(core_axis_name="c", subcore_axis_name="s"),
        scratch_types=[
            pltpu.VMEM((_CHUNK,), jnp.int32),
            pltpu.VMEM((_CHUNK, 160), jnp.float32),
            pltpu.VMEM_SHARED((N_NODES, 160), jnp.float32),
        ],
        compiler_params=pltpu.CompilerParams(use_tc_tiling_on_sc=False),
    )
    return f(ea, eb, ec, recv2d, zer)


def _sym_body(nf_ref, out_ref):
    h0 = nf_ref[0]
    h1 = nf_ref[1]
    nu1 = h0[:, 0:32]
    nu21 = (h0[:, 32:64] * h0[:, 32:64]
            + h0[:, 64:96] * h0[:, 64:96]
            + h0[:, 96:128] * h0[:, 96:128])
    a4 = h0[:, 128:160]
    nu22 = _PREF2[0] * a4 * a4
    for j, pref in enumerate(_PREF2[1:]):
        blk = h1[:, 32 * j:32 * j + 32]
        nu22 = nu22 + pref * blk * blk
    pieces = []
    for s in range(8):
        pieces.append(nu1[:, 4 * s:4 * s + 4])
        pieces.append(nu21[:, 4 * s:4 * s + 4])
        pieces.append(nu22[:, 4 * s:4 * s + 4])
    out_ref[...] = jnp.concatenate(pieces, axis=1)


def _symmetrize(nfa):
    grid = N_NODES // _NB
    return pl.pallas_call(
        _sym_body,
        grid=(grid,),
        in_specs=[pl.BlockSpec((2, _NB, 160), lambda i: (0, i, 0))],
        out_specs=pl.BlockSpec((_NB, 96), lambda i: (i, 0)),
        out_shape=jax.ShapeDtypeStruct((N_NODES, 96), jnp.float32),
    )(nfa)


def kernel(positions, atomic_numbers, edge_index, shifts,
           node_embedding_W, radial_transform_W):
    n = positions.shape[0]
    z2d = atomic_numbers.reshape(n, 1).astype(jnp.int32)
    table = _build_table(positions, z2d, node_embedding_W)

    idx2d = edge_index.astype(jnp.int32).reshape(2 * _N_CHUNKS, _CHUNK)
    gathered = _sc_gather(table, idx2d)

    ea, eb, ec = _edge_expand(gathered, shifts.T, radial_transform_W)

    recv2d = edge_index[1].astype(jnp.int32).reshape(_N_CHUNKS, _CHUNK)
    zer = jnp.zeros((_STRIPE, 160), jnp.float32)
    nfa = _sc_scatter(ea, eb, ec, recv2d, zer)

    out96 = _symmetrize(nfa)
    return out96.reshape(n, 8, 3, 4)
